# baseline (speedup 1.0000x reference)
import functools

import jax
import jax.numpy as jnp
from jax import lax
from jax.experimental import pallas as pl
from jax.experimental.pallas import tpu as pltpu
from jax.experimental.pallas import tpu_sc as plsc

BATCH = 16384
SDIM = 32
IDIM = 32
DDIM = 8

NC = 2
NS = 16
NW = NC * NS
B_PER_W = BATCH // NW
N_LT = B_PER_W // 128
GROUPS_PER_LT = 128 // 16
CHUNK = 128
N_CHUNK = B_PER_W // CHUNK


def _build_student():
    mesh = plsc.VectorSubcoreMesh(core_axis_name="c", subcore_axis_name="s")

    @functools.partial(
        pl.kernel,
        mesh=mesh,
        compiler_params=pltpu.CompilerParams(needs_layout_passes=False),
        out_type=jax.ShapeDtypeStruct((SDIM, BATCH), jnp.float32),
        scratch_types=[
            pltpu.VMEM((B_PER_W,), jnp.int32),
            pltpu.VMEM((32,), jnp.int32),
            pltpu.VMEM((4, SDIM, 128), jnp.float32),
            pltpu.VMEM((SDIM, 128), jnp.float32),
            pltpu.SemaphoreType.DMA,
        ],
    )
    def stu_kernel(stu_idx_hbm, stu_tab, stu_out,
                   sidx_v, riota_v, sfetch, sstage, sem_s):
        wid = lax.axis_index("s") * NC + lax.axis_index("c")
        base = wid * B_PER_W
        pltpu.sync_copy(stu_idx_hbm.at[pl.ds(base, B_PER_W)], sidx_v)

        rows = lax.iota(jnp.int32, 16)
        riota_v[pl.ds(0, 16)] = rows
        riota_v[pl.ds(16, 16)] = rows + 16

        def fetch(cvec, t, slot):
            w_s = pl.multiple_of((cvec[t] >> 7) << 7, 128)
            return pltpu.async_copy(
                stu_tab.at[riota_v, pl.ds(w_s, 128)], sfetch.at[slot], sem_s)

        def extract(cvec, t, slot, lane_i):
            qs = jnp.full((16,), cvec[t] & 127, jnp.int32)
            lane = jnp.full((16,), lane_i, jnp.int32)
            v0 = plsc.load_gather(sfetch.at[slot], [rows, qs])
            v1 = plsc.load_gather(sfetch.at[slot], [rows + 16, qs])
            plsc.store_scatter(sstage, [rows, lane], v0)
            plsc.store_scatter(sstage, [rows + 16, lane], v1)

        for jj in range(N_LT):
            def body(g, _):
                cvec = sidx_v[pl.ds(jj * 128 + g * 16, 16)]
                pipe = [fetch(cvec, 0, 0), fetch(cvec, 1, 1),
                        fetch(cvec, 2, 2)]
                for t in range(16):
                    if t < 13:
                        pipe.append(fetch(cvec, t + 3, (t + 3) % 4))
                    pipe[t].wait()
                    extract(cvec, t, t % 4, g * 16 + t)
                return _

            lax.fori_loop(0, GROUPS_PER_LT, body, 0)
            lane0 = pl.multiple_of((wid * N_LT + jj) * 128, 128)
            pltpu.sync_copy(sstage, stu_out.at[:, pl.ds(lane0, 128)])

    return stu_kernel


def _build_exer():
    mesh = plsc.VectorSubcoreMesh(core_axis_name="c", subcore_axis_name="s")

    @functools.partial(
        pl.kernel,
        mesh=mesh,
        compiler_params=pltpu.CompilerParams(use_tc_tiling_on_sc=False),
        out_type=(
            jax.ShapeDtypeStruct((BATCH, IDIM), jnp.float32),
            jax.ShapeDtypeStruct((BATCH, DDIM), jnp.float32),
        ),
        scratch_types=[
            pltpu.VMEM((B_PER_W,), jnp.int32),
            pltpu.VMEM((B_PER_W, IDIM), jnp.float32),
            pltpu.VMEM((B_PER_W, DDIM), jnp.float32),
            pltpu.SemaphoreType.DMA,
            pltpu.SemaphoreType.DMA,
        ],
    )
    def exer_kernel(exer_idx_hbm, item_tab, disc_tab,
                    item_out, disc_out,
                    eidx_v, irows_v, drows_v, sem_i, sem_d):
        wid = lax.axis_index("s") * NC + lax.axis_index("c")
        base = wid * B_PER_W
        pltpu.sync_copy(exer_idx_hbm.at[pl.ds(base, B_PER_W)], eidx_v)
        waits = []
        for j in range(N_CHUNK):
            waits.append(pltpu.async_copy(
                item_tab.at[eidx_v.at[pl.ds(j * CHUNK, CHUNK)]],
                irows_v.at[pl.ds(j * CHUNK, CHUNK)], sem_i))
        for j in range(N_CHUNK):
            waits.append(pltpu.async_copy(
                disc_tab.at[eidx_v.at[pl.ds(j * CHUNK, CHUNK)]],
                drows_v.at[pl.ds(j * CHUNK, CHUNK)], sem_d))
        for w in waits[:N_CHUNK]:
            w.wait()
        pltpu.sync_copy(irows_v, item_out.at[pl.ds(base, B_PER_W)])
        for w in waits[N_CHUNK:]:
            w.wait()
        pltpu.sync_copy(drows_v, disc_out.at[pl.ds(base, B_PER_W)])

    return exer_kernel


_STU_KERNEL = _build_student()
_EXER_KERNEL = _build_exer()


def kernel(stu_idx, exer_idx, student_table, item_table, disc_table):
    item, disc = _EXER_KERNEL(exer_idx.astype(jnp.int32),
                              item_table, disc_table)
    stu_t = _STU_KERNEL(stu_idx.astype(jnp.int32), student_table.T)
    return (stu_t.T, item, disc)


# speedup vs baseline: 1.0687x; 1.0687x over previous
import functools

import jax
import jax.numpy as jnp
from jax import lax
from jax.experimental import pallas as pl
from jax.experimental.pallas import tpu as pltpu
from jax.experimental.pallas import tpu_sc as plsc

BATCH = 16384
SDIM = 32
IDIM = 32
DDIM = 8

NC = 2
NS = 16
NW = NC * NS
B_PER_W = BATCH // NW
N_LT = B_PER_W // 128
GROUPS_PER_LT = 128 // 16
CHUNK = 128
N_CHUNK = B_PER_W // CHUNK


def _build_student():
    mesh = plsc.VectorSubcoreMesh(core_axis_name="c", subcore_axis_name="s")

    @functools.partial(
        pl.kernel,
        mesh=mesh,
        compiler_params=pltpu.CompilerParams(needs_layout_passes=False),
        out_type=jax.ShapeDtypeStruct((SDIM, BATCH), jnp.float32),
        scratch_types=[
            pltpu.VMEM((B_PER_W,), jnp.int32),
            pltpu.VMEM((32,), jnp.int32),
            pltpu.VMEM((4, SDIM, 128), jnp.float32),
            pltpu.VMEM((SDIM, B_PER_W), jnp.float32),
            pltpu.SemaphoreType.DMA,
        ],
    )
    def stu_kernel(stu_idx_hbm, stu_tab, stu_out,
                   sidx_v, riota_v, sfetch, sstage, sem_s):
        wid = lax.axis_index("s") * NC + lax.axis_index("c")
        base = wid * B_PER_W
        pltpu.sync_copy(stu_idx_hbm.at[pl.ds(base, B_PER_W)], sidx_v)

        rows = lax.iota(jnp.int32, 16)
        riota_v[pl.ds(0, 16)] = rows
        riota_v[pl.ds(16, 16)] = rows + 16

        def fetch(cvec, t, slot):
            w_s = pl.multiple_of((cvec[t] >> 7) << 7, 128)
            return pltpu.async_copy(
                stu_tab.at[riota_v, pl.ds(w_s, 128)], sfetch.at[slot], sem_s)

        def drain():
            return pltpu.make_async_copy(
                stu_tab.at[riota_v, pl.ds(0, 128)], sfetch.at[0], sem_s)

        def extract(cvec, t, slot, lane_i):
            qs = jnp.full((16,), cvec[t] & 127, jnp.int32)
            lane = jnp.full((16,), lane_i, jnp.int32)
            v0 = plsc.load_gather(sfetch.at[slot], [rows, qs])
            v1 = plsc.load_gather(sfetch.at[slot], [rows + 16, qs])
            plsc.store_scatter(sstage, [rows, lane], v0)
            plsc.store_scatter(sstage, [rows + 16, lane], v1)

        cvec0 = sidx_v[pl.ds(0, 16)]
        for t in range(3):
            fetch(cvec0, t, t)

        def body(g, _):
            off = g * 16
            cvec = sidx_v[pl.ds(off, 16)]
            nvec = sidx_v[pl.ds(off + 16, 16)]
            for t in range(16):
                p = off + t
                if t < 13:
                    fetch(cvec, t + 3, (t + 3) % 4)
                else:
                    fetch(nvec, t - 13, (t + 3) % 4)
                drain().wait()
                extract(cvec, t, t % 4, p)
            return _

        lax.fori_loop(0, B_PER_W // 16 - 1, body, 0)

        off = B_PER_W - 16
        cvecz = sidx_v[pl.ds(off, 16)]
        for t in range(16):
            if t < 13:
                fetch(cvecz, t + 3, (t + 3) % 4)
            drain().wait()
            extract(cvecz, t, (off + t) % 4, off + t)

        lane0 = pl.multiple_of(wid * B_PER_W, 128)
        pltpu.sync_copy(sstage, stu_out.at[:, pl.ds(lane0, B_PER_W)])

    return stu_kernel


def _build_exer():
    mesh = plsc.VectorSubcoreMesh(core_axis_name="c", subcore_axis_name="s")

    @functools.partial(
        pl.kernel,
        mesh=mesh,
        compiler_params=pltpu.CompilerParams(use_tc_tiling_on_sc=False),
        out_type=(
            jax.ShapeDtypeStruct((BATCH, IDIM), jnp.float32),
            jax.ShapeDtypeStruct((BATCH, DDIM), jnp.float32),
        ),
        scratch_types=[
            pltpu.VMEM((B_PER_W,), jnp.int32),
            pltpu.VMEM((B_PER_W, IDIM), jnp.float32),
            pltpu.VMEM((B_PER_W, DDIM), jnp.float32),
            pltpu.SemaphoreType.DMA,
            pltpu.SemaphoreType.DMA,
        ],
    )
    def exer_kernel(exer_idx_hbm, item_tab, disc_tab,
                    item_out, disc_out,
                    eidx_v, irows_v, drows_v, sem_i, sem_d):
        wid = lax.axis_index("s") * NC + lax.axis_index("c")
        base = wid * B_PER_W
        pltpu.sync_copy(exer_idx_hbm.at[pl.ds(base, B_PER_W)], eidx_v)
        waits = []
        for j in range(N_CHUNK):
            waits.append(pltpu.async_copy(
                item_tab.at[eidx_v.at[pl.ds(j * CHUNK, CHUNK)]],
                irows_v.at[pl.ds(j * CHUNK, CHUNK)], sem_i))
        for j in range(N_CHUNK):
            waits.append(pltpu.async_copy(
                disc_tab.at[eidx_v.at[pl.ds(j * CHUNK, CHUNK)]],
                drows_v.at[pl.ds(j * CHUNK, CHUNK)], sem_d))
        for w in waits[:N_CHUNK]:
            w.wait()
        pltpu.sync_copy(irows_v, item_out.at[pl.ds(base, B_PER_W)])
        for w in waits[N_CHUNK:]:
            w.wait()
        pltpu.sync_copy(drows_v, disc_out.at[pl.ds(base, B_PER_W)])

    return exer_kernel


_STU_KERNEL = _build_student()
_EXER_KERNEL = _build_exer()


def kernel(stu_idx, exer_idx, student_table, item_table, disc_table):
    item, disc = _EXER_KERNEL(exer_idx.astype(jnp.int32),
                              item_table, disc_table)
    stu_t = _STU_KERNEL(stu_idx.astype(jnp.int32), student_table.T)
    return (stu_t.T, item, disc)
